# Initial kernel scaffold; baseline (speedup 1.0000x reference)
import jax
import jax.numpy as jnp
from jax.experimental import pallas as pl


def kernel(pts, viewdirs, iteration, embedder_position, embedder_viewdir, xvec, yvec, zvec, tvec):
    raise NotImplementedError("write your pallas kernel here")


# speedup vs baseline: 2.2730x; 2.2730x over previous
import functools

import jax
import jax.numpy as jnp
from jax import lax
from jax.experimental import pallas as pl
from jax.experimental.pallas import tpu as pltpu
from jax.experimental.pallas import tpu_sc as plsc

_N = 131072
_FCH = 4
_R = 96
_C = _FCH * _R
_NG = 2048
_NT = 128
_NCORE = 2
_NSUB = 16
_NW = _NCORE * _NSUB
_PPW = _N // _NW
_P = 32
_NCHUNK = _PPW // _P
_L = 16
_NROW = _FCH * _P

_GRIDS = (_NG, _NG, _NG, _NT)
_WSTR = _P + _L


def _tec_body(px_h, py_h, pz_h, pt_h, xov, yov, zov, tov, col_o, sig_o,
              pts_v, idx_v, w1f, xr_v, yr_v, zr_v, tr_v,
              accs, tmp_v, ptmp_v, btmp_v, col_st, sig_st, sem):
    wid = lax.axis_index("s") * _NCORE + lax.axis_index("c")
    base = wid * _PPW
    lanes = lax.iota(jnp.int32, _L)
    zero16 = jnp.zeros((_L,), jnp.float32)

    for r, col in enumerate((px_h, py_h, pz_h, pt_h)):
        pltpu.sync_copy(col.at[pl.ds(base, _PPW)], pts_v.at[r])

    for q in range(8):
        ptmp_v[pl.ds(q * 48, _L)] = zero16
        ptmp_v[pl.ds(q * 48 + 32, _L)] = zero16
    for k in range(4):
        btmp_v[pl.ds(k * 48, _L)] = zero16

    tables = (xov, yov, zov, tov)
    rows = (xr_v, yr_v, zr_v, tr_v)

    def chunk_body(g, _):
        for u in range(_P // _L):
            off = g * _P + u * _L
            cx = pts_v[0, pl.ds(off, _L)] / 1.0
            cy = (pts_v[1, pl.ds(off, _L)] - (-1.0)) / 2.0 * 2.0 - 1.0
            cz = pts_v[2, pl.ds(off, _L)] / 1.0
            ct = pts_v[3, pl.ds(off, _L)] * 2.0 - 1.0
            for k, coord in enumerate((cx, cy, cz, ct)):
                nc = _GRIDS[k]
                iy = ((coord + 1.0) * nc - 1.0) / 2.0
                i0t = iy.astype(jnp.int32)
                i0 = jnp.where(iy < i0t.astype(jnp.float32), i0t - 1, i0t)
                w1 = iy - i0.astype(jnp.float32)
                j = jnp.clip(i0 + 1, 0, nc)
                idx_v[k, pl.ds(u * _L, _L)] = j
                w1f[pl.ds(k * _WSTR + u * _L, _L)] = w1

        dmas = [pltpu.async_copy(tables[k].at[idx_v.at[k]], rows[k], sem)
                for k in range(4)]
        for d in dmas:
            d.wait()

        def point_body(p, _):
            w1b, w0b = [], []
            for k in range(4):
                v = w1f[pl.ds(k * _WSTR + p, _L)]
                s = jnp.where(lanes == 0, v, 0.0)
                tb = k * 48
                for shift in (15, 14, 12, 8):
                    btmp_v[pl.ds(tb + _L, _L)] = s
                    s = s + btmp_v[pl.ds(tb + shift, _L)]
                w1b.append(s)
                w0b.append(1.0 - s)
            for j in range(_C // _L):
                prod = None
                for k in range(4):
                    g0 = rows[k][p, pl.ds(j * _L, _L)]
                    g1 = rows[k][p, pl.ds(_C + j * _L, _L)]
                    a = g0 * w0b[k] + g1 * w1b[k]
                    prod = a if prod is None else prod * a
                ch = j // (_R // _L)
                sl = pl.ds(p * _FCH * _L + ch * _L, _L)
                if j % (_R // _L) == 0:
                    accs[sl] = prod
                else:
                    accs[sl] = accs[sl] + prod
            return ()

        lax.fori_loop(0, _P, point_body, (), unroll=False)

        chacc = [[zero16 for _ in range(_FCH)] for _ in range(_P // _L)]
        for r in range(_NROW):
            p, ch = r // _FCH, r % _FCH
            u, t = p // _L, p % _L
            fb = r * _L
            q = (r % 8) * 64
            s = accs[pl.ds(fb, _L)] + accs[pl.ds(fb + 8, _L)]
            for shift in (20, 18, 17):
                tmp_v[pl.ds(q + _L, _L)] = s
                s = s + tmp_v[pl.ds(q + shift, _L)]
            m = jnp.where(lanes == 0, s, 0.0)
            rq = (r % 8) * 48
            ptmp_v[pl.ds(rq + _L, _L)] = m
            placed = ptmp_v[pl.ds(rq + _L - t, _L)]
            chacc[u][ch] = chacc[u][ch] + placed

        for u in range(_P // _L):
            off = g * _P + u * _L
            x = pts_v[0, pl.ds(off, _L)]
            y = pts_v[1, pl.ds(off, _L)]
            z = pts_v[2, pl.ds(off, _L)]
            inside = ((x * x + z * z <= 1.0) & (y >= -1.0) & (y <= 1.0))
            sig_st[pl.ds(u * _L, _L)] = jnp.where(
                inside, jnp.exp(chacc[u][0]), 0.0)
            for c in range(1, _FCH):
                col_st[pl.ds((c - 1) * _P + u * _L, _L)] = (
                    1.0 / (1.0 + jnp.exp(-chacc[u][c])))

        pltpu.sync_copy(sig_st, sig_o.at[pl.ds(base + g * _P, _P)])
        for c in range(1, _FCH):
            pltpu.sync_copy(
                col_st.at[pl.ds((c - 1) * _P, _P)],
                col_o.at[pl.ds((c - 1) * _N + base + g * _P, _P)])
        return ()

    lax.fori_loop(0, _NCHUNK, chunk_body, (), unroll=False)


@functools.partial(
    pl.kernel,
    out_type=[jax.ShapeDtypeStruct((_N * 3,), jnp.float32),
              jax.ShapeDtypeStruct((_N,), jnp.float32)],
    mesh=plsc.VectorSubcoreMesh(core_axis_name="c", subcore_axis_name="s"),
    scratch_types=[
        pltpu.VMEM((4, _PPW), jnp.float32),
        pltpu.VMEM((4, _P), jnp.int32),
        pltpu.VMEM((4 * _WSTR,), jnp.float32),
        pltpu.VMEM((_P, 2 * _C), jnp.float32),
        pltpu.VMEM((_P, 2 * _C), jnp.float32),
        pltpu.VMEM((_P, 2 * _C), jnp.float32),
        pltpu.VMEM((_P, 2 * _C), jnp.float32),
        pltpu.VMEM((_NROW * _L + _L,), jnp.float32),
        pltpu.VMEM((8 * 64,), jnp.float32),
        pltpu.VMEM((8 * 48,), jnp.float32),
        pltpu.VMEM((4 * 48,), jnp.float32),
        pltpu.VMEM((3 * _P,), jnp.float32),
        pltpu.VMEM((_P,), jnp.float32),
        pltpu.SemaphoreType.DMA,
    ],
)
def _sc_lookup(px_h, py_h, pz_h, pt_h, xov, yov, zov, tov, col_o, sig_o,
               *scratch):
    _tec_body(px_h, py_h, pz_h, pt_h, xov, yov, zov, tov, col_o, sig_o,
              *scratch)


def _prep_table(vec):
    v = vec[0, :, :, 0].T
    z = jnp.zeros((1, _C), v.dtype)
    vp = jnp.concatenate([z, v, z], axis=0)
    return jnp.concatenate([vp[:-1], vp[1:]], axis=1)


def kernel(pts, viewdirs, iteration, embedder_position, embedder_viewdir,
           xvec, yvec, zvec, tvec):
    del viewdirs, iteration, embedder_position, embedder_viewdir
    col_flat, sigma = _sc_lookup(
        pts[:, 0], pts[:, 1], pts[:, 2], pts[:, 3],
        _prep_table(xvec), _prep_table(yvec),
        _prep_table(zvec), _prep_table(tvec))
    return col_flat.reshape(3, _N).T, sigma


# speedup vs baseline: 3.2704x; 1.4388x over previous
import functools

import jax
import jax.numpy as jnp
from jax import lax
from jax.experimental import pallas as pl
from jax.experimental.pallas import tpu as pltpu
from jax.experimental.pallas import tpu_sc as plsc

_N = 131072
_FCH = 4
_R = 96
_C = _FCH * _R
_NG = 2048
_NT = 128
_NCORE = 2
_NSUB = 16
_NW = _NCORE * _NSUB
_PPW = _N // _NW
_P = 32
_NCHUNK = _PPW // _P
_L = 16
_NROW = _FCH * _P

_GRIDS = (_NG, _NG, _NG, _NT)
_WSTR = _P + _L


def _tec_body(px_h, py_h, pz_h, pt_h, xov, yov, zov, tov, col_o, sig_o,
              pts_v, idx_v, w1f, xr_v, yr_v, zr_v, tr_v,
              accs, tmp_v, ptmp_v, btmp_v, col_st, sig_st, sem):
    wid = lax.axis_index("s") * _NCORE + lax.axis_index("c")
    base = wid * _PPW
    lanes = lax.iota(jnp.int32, _L)
    zero16 = jnp.zeros((_L,), jnp.float32)

    for r, col in enumerate((px_h, py_h, pz_h, pt_h)):
        pltpu.sync_copy(col.at[pl.ds(base, _PPW)], pts_v.at[r])

    for q in range(8):
        ptmp_v[pl.ds(q * 48, _L)] = zero16
        ptmp_v[pl.ds(q * 48 + 32, _L)] = zero16
    for k in range(4):
        btmp_v[pl.ds(k * 48, _L)] = zero16

    tables = (xov, yov, zov, tov)
    rows = (xr_v, yr_v, zr_v, tr_v)

    def chunk_body(g, _):
        for u in range(_P // _L):
            off = g * _P + u * _L
            cx = pts_v[0, pl.ds(off, _L)] / 1.0
            cy = (pts_v[1, pl.ds(off, _L)] - (-1.0)) / 2.0 * 2.0 - 1.0
            cz = pts_v[2, pl.ds(off, _L)] / 1.0
            ct = pts_v[3, pl.ds(off, _L)] * 2.0 - 1.0
            for k, coord in enumerate((cx, cy, cz, ct)):
                nc = _GRIDS[k]
                iy = ((coord + 1.0) * nc - 1.0) / 2.0
                i0t = iy.astype(jnp.int32)
                i0 = jnp.where(iy < i0t.astype(jnp.float32), i0t - 1, i0t)
                w1 = iy - i0.astype(jnp.float32)
                j = jnp.clip(i0 + 1, 0, nc)
                idx_v[k, pl.ds(u * _L, _L)] = j
                w1f[pl.ds(k * _WSTR + u * _L, _L)] = w1

        dmas = [pltpu.async_copy(tables[k].at[idx_v.at[k]], rows[k], sem)
                for k in range(4)]
        for d in dmas:
            d.wait()

        def point_body(p, _):
            w1b, w0b = [], []
            for k in range(4):
                v = w1f[pl.ds(k * _WSTR + p, _L)]
                s = jnp.where(lanes == 0, v, 0.0)
                tb = k * 48
                for shift in (15, 14, 12, 8):
                    btmp_v[pl.ds(tb + _L, _L)] = s
                    s = s + btmp_v[pl.ds(tb + shift, _L)]
                w1b.append(s)
                w0b.append(1.0 - s)
            accv = [None] * _FCH
            for j in range(_C // _L):
                prod = None
                for k in range(4):
                    g0 = rows[k][p, pl.ds(j * _L, _L)]
                    g1 = rows[k][p, pl.ds(_C + j * _L, _L)]
                    a = g0 * w0b[k] + g1 * w1b[k]
                    prod = a if prod is None else prod * a
                ch = j // (_R // _L)
                accv[ch] = prod if accv[ch] is None else accv[ch] + prod
            for ch in range(_FCH):
                accs[pl.ds(p * _FCH * _L + ch * _L, _L)] = accv[ch]
            return ()

        lax.fori_loop(0, _P, point_body, (), unroll=False)

        chacc = [[zero16 for _ in range(_FCH)] for _ in range(_P // _L)]
        for r in range(_NROW):
            p, ch = r // _FCH, r % _FCH
            u, t = p // _L, p % _L
            fb = r * _L
            q = (r % 8) * 64
            s = accs[pl.ds(fb, _L)] + accs[pl.ds(fb + 8, _L)]
            for shift in (20, 18, 17):
                tmp_v[pl.ds(q + _L, _L)] = s
                s = s + tmp_v[pl.ds(q + shift, _L)]
            m = jnp.where(lanes == 0, s, 0.0)
            rq = (r % 8) * 48
            ptmp_v[pl.ds(rq + _L, _L)] = m
            placed = ptmp_v[pl.ds(rq + _L - t, _L)]
            chacc[u][ch] = chacc[u][ch] + placed

        for u in range(_P // _L):
            off = g * _P + u * _L
            x = pts_v[0, pl.ds(off, _L)]
            y = pts_v[1, pl.ds(off, _L)]
            z = pts_v[2, pl.ds(off, _L)]
            inside = ((x * x + z * z <= 1.0) & (y >= -1.0) & (y <= 1.0))
            sig_st[pl.ds(u * _L, _L)] = jnp.where(
                inside, jnp.exp(chacc[u][0]), 0.0)
            for c in range(1, _FCH):
                col_st[pl.ds((c - 1) * _P + u * _L, _L)] = (
                    1.0 / (1.0 + jnp.exp(-chacc[u][c])))

        pltpu.sync_copy(sig_st, sig_o.at[pl.ds(base + g * _P, _P)])
        for c in range(1, _FCH):
            pltpu.sync_copy(
                col_st.at[pl.ds((c - 1) * _P, _P)],
                col_o.at[pl.ds((c - 1) * _N + base + g * _P, _P)])
        return ()

    lax.fori_loop(0, _NCHUNK, chunk_body, (), unroll=False)


@functools.partial(
    pl.kernel,
    out_type=[jax.ShapeDtypeStruct((_N * 3,), jnp.float32),
              jax.ShapeDtypeStruct((_N,), jnp.float32)],
    mesh=plsc.VectorSubcoreMesh(core_axis_name="c", subcore_axis_name="s"),
    scratch_types=[
        pltpu.VMEM((4, _PPW), jnp.float32),
        pltpu.VMEM((4, _P), jnp.int32),
        pltpu.VMEM((4 * _WSTR,), jnp.float32),
        pltpu.VMEM((_P, 2 * _C), jnp.float32),
        pltpu.VMEM((_P, 2 * _C), jnp.float32),
        pltpu.VMEM((_P, 2 * _C), jnp.float32),
        pltpu.VMEM((_P, 2 * _C), jnp.float32),
        pltpu.VMEM((_NROW * _L + _L,), jnp.float32),
        pltpu.VMEM((8 * 64,), jnp.float32),
        pltpu.VMEM((8 * 48,), jnp.float32),
        pltpu.VMEM((4 * 48,), jnp.float32),
        pltpu.VMEM((3 * _P,), jnp.float32),
        pltpu.VMEM((_P,), jnp.float32),
        pltpu.SemaphoreType.DMA,
    ],
)
def _sc_lookup(px_h, py_h, pz_h, pt_h, xov, yov, zov, tov, col_o, sig_o,
               *scratch):
    _tec_body(px_h, py_h, pz_h, pt_h, xov, yov, zov, tov, col_o, sig_o,
              *scratch)


def _prep_table(vec):
    v = vec[0, :, :, 0].T
    z = jnp.zeros((1, _C), v.dtype)
    vp = jnp.concatenate([z, v, z], axis=0)
    return jnp.concatenate([vp[:-1], vp[1:]], axis=1)


def kernel(pts, viewdirs, iteration, embedder_position, embedder_viewdir,
           xvec, yvec, zvec, tvec):
    del viewdirs, iteration, embedder_position, embedder_viewdir
    col_flat, sigma = _sc_lookup(
        pts[:, 0], pts[:, 1], pts[:, 2], pts[:, 3],
        _prep_table(xvec), _prep_table(yvec),
        _prep_table(zvec), _prep_table(tvec))
    return col_flat.reshape(3, _N).T, sigma
